# baseline (speedup 1.0000x reference)
import functools

import jax
import jax.numpy as jnp
from jax import lax
from jax.experimental import pallas as pl
from jax.experimental.pallas import tpu as pltpu
from jax.experimental.pallas import tpu_sc as plsc

_VOCAB = 1000
_D = 64
_B = 16384
_CTX = 10
_NEG = 6
_R = _CTX + _NEG

_NC = 2
_NS = 16
_L = 16
_NW = _NC * _NS
_BPW = _B // _NW
_G = _BPW // _L
_CPG = _L * _CTX
_RPG = _L * _R
_KD = _D // _L

_mesh = plsc.VectorSubcoreMesh(core_axis_name="c", subcore_axis_name="s")


@functools.partial(
    pl.kernel,
    out_type=jax.ShapeDtypeStruct((_B * _NEG,), jnp.float32),
    mesh=_mesh,
    scratch_types=[
        pltpu.VMEM((_BPW * _R,), jnp.int32),
        pltpu.VMEM((_RPG, _D), jnp.float32),
        pltpu.VMEM((_RPG, _D), jnp.float32),
        pltpu.VMEM((_BPW * _NEG + _L,), jnp.float32),
        pltpu.SemaphoreType.DMA,
        pltpu.SemaphoreType.DMA,
    ],
    compiler_params=pltpu.CompilerParams(needs_layout_passes=False,
                                         disable_bounds_checks=True,
                                         use_tc_tiling_on_sc=False),
)
def _cbow(idx_hbm, w_hbm, out_hbm,
          idx_v, buf_a, buf_b, out_v, sem_a, sem_b):
    wid = lax.axis_index("s") * _NC + lax.axis_index("c")
    pltpu.sync_copy(idx_hbm.at[pl.ds(wid * (_BPW * _R), _BPW * _R)], idx_v)

    iota = lax.iota(jnp.int32, _L)
    last_mask = iota == (_L - 1)

    bufs = (buf_a, buf_b)
    sems = (sem_a, sem_b)

    def fire(g, b):
        for h in range(2):
            pltpu.async_copy(
                w_hbm.at[idx_v.at[pl.ds(g * _RPG + h * 128, 128)]],
                bufs[b].at[pl.ds(h * 128, 128)], sems[b])

    def drain(b):
        for h in range(2):
            pltpu.make_async_copy(
                w_hbm.at[idx_v.at[pl.ds(h * 128, 128)]],
                bufs[b].at[pl.ds(h * 128, 128)], sems[b]).wait()

    def compute(g, b):
        buf = bufs[b]
        for e in range(_L):
            crows = e * _R
            acc = []
            for k in range(_KD):
                cx = [buf[crows + c, pl.ds(k * _L, _L)] for c in range(_CTX)]
                acc.append((((cx[0] + cx[1]) + (cx[2] + cx[3]))
                            + ((cx[4] + cx[5]) + (cx[6] + cx[7]))
                            + (cx[8] + cx[9])))
            for j in range(_NEG):
                nr = e * _R + _CTX + j
                p01 = (buf[nr, pl.ds(0, _L)] * acc[0]
                       + buf[nr, pl.ds(_L, _L)] * acc[1])
                p23 = (buf[nr, pl.ds(2 * _L, _L)] * acc[2]
                       + buf[nr, pl.ds(3 * _L, _L)] * acc[3])
                plsc.store_compressed(
                    out_v.at[pl.ds(j * _BPW + g * _L + e, _L)],
                    plsc.cumsum(p01 + p23), mask=last_mask)
        logits = [out_v[pl.ds(j * _BPW + g * _L, _L)] for j in range(_NEG)]
        m = logits[0]
        for j in range(1, _NEG):
            m = jnp.maximum(m, logits[j])
        es = [jnp.exp(l - m) for l in logits]
        tot = es[0]
        for j in range(1, _NEG):
            tot = tot + es[j]
        for j in range(_NEG):
            out_v[pl.ds(j * _BPW + g * _L, _L)] = es[j] / tot

    fire(0, 0)

    def body(i, carry):
        g0 = i * 2
        fire(g0 + 1, 1)
        drain(0)
        compute(g0, 0)

        @pl.when(i < _G // 2 - 1)
        def _():
            fire(g0 + 2, 0)

        drain(1)
        compute(g0 + 1, 1)
        return carry

    lax.fori_loop(0, _G // 2, body, 0)
    pltpu.sync_copy(out_v.at[pl.ds(0, _BPW * _NEG)],
                    out_hbm.at[pl.ds(wid * (_BPW * _NEG), _BPW * _NEG)])


def kernel(input_words, negative_samples, W):
    idx_all = jnp.concatenate([input_words, negative_samples], axis=1)
    out_t = _cbow(idx_all.reshape(-1), W)
    return (out_t.reshape(_NW, _NEG, _BPW)
            .transpose(0, 2, 1)
            .reshape(_B, _NEG))


# speedup vs baseline: 1.0441x; 1.0441x over previous
import functools

import jax
import jax.numpy as jnp
from jax import lax
from jax.experimental import pallas as pl
from jax.experimental.pallas import tpu as pltpu
from jax.experimental.pallas import tpu_sc as plsc

_VOCAB = 1000
_D = 64
_B = 16384
_CTX = 10
_NEG = 6
_R = _CTX + _NEG

_NC = 2
_NS = 16
_L = 16
_NW = _NC * _NS
_BPW = _B // _NW
_G = _BPW // _L
_CPG = _L * _CTX
_RPG = _L * _R
_KD = _D // _L

_mesh = plsc.VectorSubcoreMesh(core_axis_name="c", subcore_axis_name="s")


@functools.partial(
    pl.kernel,
    out_type=jax.ShapeDtypeStruct((_B * _NEG,), jnp.float32),
    mesh=_mesh,
    scratch_types=[
        pltpu.VMEM((_BPW * _R,), jnp.int32),
        pltpu.VMEM((_RPG, _D), jnp.float32),
        pltpu.VMEM((_RPG, _D), jnp.float32),
        pltpu.VMEM((_BPW * _NEG,), jnp.float32),
        pltpu.SemaphoreType.DMA,
        pltpu.SemaphoreType.DMA,
    ],
    compiler_params=pltpu.CompilerParams(needs_layout_passes=False,
                                         disable_bounds_checks=True,
                                         use_tc_tiling_on_sc=False),
)
def _cbow(idx_hbm, w_hbm, out_hbm,
          idx_v, buf_a, buf_b, out_v, sem_a, sem_b):
    wid = lax.axis_index("s") * _NC + lax.axis_index("c")
    pltpu.sync_copy(idx_hbm.at[pl.ds(wid * (_BPW * _R), _BPW * _R)], idx_v)

    iota = lax.iota(jnp.int32, _L)
    lane_masks = [iota == e for e in range(_L)]

    bufs = (buf_a, buf_b)
    sems = (sem_a, sem_b)

    def fire(g, b):
        for h in range(2):
            pltpu.async_copy(
                w_hbm.at[idx_v.at[pl.ds(g * _RPG + h * 128, 128)]],
                bufs[b].at[pl.ds(h * 128, 128)], sems[b])

    def drain(b):
        for h in range(2):
            pltpu.make_async_copy(
                w_hbm.at[idx_v.at[pl.ds(h * 128, 128)]],
                bufs[b].at[pl.ds(h * 128, 128)], sems[b]).wait()

    def compute(g, b):
        buf = bufs[b]
        logits = [jnp.zeros((_L,), jnp.float32) for _ in range(_NEG)]
        pend = None
        for e in range(_L):
            crows = e * _R
            acc = []
            for k in range(_KD):
                cx = [buf[crows + c, pl.ds(k * _L, _L)] for c in range(_CTX)]
                acc.append((((cx[0] + cx[1]) + (cx[2] + cx[3]))
                            + ((cx[4] + cx[5]) + (cx[6] + cx[7]))
                            + (cx[8] + cx[9])))
            cs = []
            for j in range(_NEG):
                nr = e * _R + _CTX + j
                p01 = (buf[nr, pl.ds(0, _L)] * acc[0]
                       + buf[nr, pl.ds(_L, _L)] * acc[1])
                p23 = (buf[nr, pl.ds(2 * _L, _L)] * acc[2]
                       + buf[nr, pl.ds(3 * _L, _L)] * acc[3])
                cs.append(plsc.cumsum(p01 + p23))
            if pend is not None:
                pe, pcs = pend
                for j in range(_NEG):
                    logits[j] = jnp.where(lane_masks[pe], pcs[j][_L - 1],
                                          logits[j])
            pend = (e, cs)
        pe, pcs = pend
        for j in range(_NEG):
            logits[j] = jnp.where(lane_masks[pe], pcs[j][_L - 1], logits[j])
        m = logits[0]
        for j in range(1, _NEG):
            m = jnp.maximum(m, logits[j])
        es = [jnp.exp(l - m) for l in logits]
        tot = es[0]
        for j in range(1, _NEG):
            tot = tot + es[j]
        for j in range(_NEG):
            out_v[pl.ds(j * _BPW + g * _L, _L)] = es[j] / tot

    fire(0, 0)

    def body(i, carry):
        g0 = i * 2
        fire(g0 + 1, 1)
        drain(0)
        compute(g0, 0)

        @pl.when(i < _G // 2 - 1)
        def _():
            fire(g0 + 2, 0)

        drain(1)
        compute(g0 + 1, 1)
        return carry

    lax.fori_loop(0, _G // 2, body, 0)
    pltpu.sync_copy(out_v, out_hbm.at[pl.ds(wid * (_BPW * _NEG), _BPW * _NEG)])


def kernel(input_words, negative_samples, W):
    idx_all = jnp.concatenate([input_words, negative_samples], axis=1)
    out_t = _cbow(idx_all.reshape(-1), W)
    return (out_t.reshape(_NW, _NEG, _BPW)
            .transpose(0, 2, 1)
            .reshape(_B, _NEG))
